# baseline (speedup 1.0000x reference)
import functools
import math

import jax
import jax.numpy as jnp
from jax import lax
from jax.experimental import pallas as pl
from jax.experimental.pallas import tpu as pltpu

_E = 8
_TOP_K = 2
_D = 1024
_F = 4096
_CAP_FACTOR = 1.25

_FT = 512


def _mlp_body(buf_ref, w1_ref, b1_ref, w2_ref, b2_ref, out_ref):
    f = pl.program_id(1)
    x = buf_ref[0].astype(jnp.bfloat16)
    h = jnp.dot(x, w1_ref[0].astype(jnp.bfloat16),
                preferred_element_type=jnp.float32)
    h = h + b1_ref[0, 0]
    h = 0.5 * h * (1.0 + lax.erf(h * (1.0 / math.sqrt(2.0))))
    p = jnp.dot(h.astype(jnp.bfloat16), w2_ref[0].astype(jnp.bfloat16),
                preferred_element_type=jnp.float32)

    @pl.when(f == 0)
    def _():
        out_ref[0] = p + b2_ref[0, 0]

    @pl.when(f > 0)
    def _():
        out_ref[0] += p


def _expert_mlp(buf, W1, b1, W2, b2, C):
    nf = _F // _FT
    return pl.pallas_call(
        _mlp_body,
        grid=(_E, nf),
        in_specs=[
            pl.BlockSpec((1, C, _D), lambda e, f: (e, 0, 0)),
            pl.BlockSpec((1, _D, _FT), lambda e, f: (e, 0, f)),
            pl.BlockSpec((1, 1, _FT), lambda e, f: (e, 0, f)),
            pl.BlockSpec((1, _FT, _D), lambda e, f: (e, f, 0)),
            pl.BlockSpec((1, 1, _D), lambda e, f: (e, 0, 0)),
        ],
        out_specs=pl.BlockSpec((1, C, _D), lambda e, f: (e, 0, 0)),
        out_shape=jax.ShapeDtypeStruct((_E, C, _D), jnp.float32),
        compiler_params=pltpu.CompilerParams(
            dimension_semantics=("parallel", "arbitrary"),
        ),
    )(buf, W1, b1[:, None, :], W2, b2[:, None, :])


def kernel(input, Wg, W1, b1, W2, b2):
    B, S, Dm = input.shape
    T = B * S
    xf = input.reshape(T, Dm)
    C = int(_CAP_FACTOR * T * _TOP_K / _E)
    logits = xf @ Wg
    gates = jax.nn.softmax(logits, axis=-1)
    topv, topi = jax.lax.top_k(gates, _TOP_K)
    denom = jnp.sum(topv, axis=-1, keepdims=True) + 1e-9
    topw = topv / denom
    e_flat = topi.T.reshape(-1)
    w_flat = topw.T.reshape(-1)
    oh = jax.nn.one_hot(e_flat, _E, dtype=jnp.int32)
    pos_in_e = jnp.cumsum(oh, axis=0) - oh
    pos = jnp.sum(pos_in_e * oh, axis=1)
    keep = pos < C
    pos_c = jnp.where(keep, pos, 0)
    keep_f = keep.astype(xf.dtype)
    x_rep = jnp.tile(xf, (_TOP_K, 1))
    vals = x_rep * keep_f[:, None]
    buf = jnp.zeros((_E, C, Dm), dtype=xf.dtype).at[e_flat, pos_c].add(vals)
    eo = _expert_mlp(buf, W1, b1, W2, b2, C)
    gathered = eo[e_flat, pos_c]
    gathered = gathered * (keep_f * w_flat)[:, None]
    y = gathered.reshape(_TOP_K, T, Dm).sum(axis=0)
    out = y.reshape(B, S, Dm)
    aux = jnp.zeros((Dm,), dtype=input.dtype)
    return (out, aux)


# speedup vs baseline: 1.1084x; 1.1084x over previous
import functools
import math

import jax
import jax.numpy as jnp
from jax import lax
from jax.experimental import pallas as pl
from jax.experimental.pallas import tpu as pltpu

_E = 8
_TOP_K = 2
_D = 1024
_F = 4096
_CAP_FACTOR = 1.25

_FT = 1024


def _mlp_body(buf_ref, w1_ref, b1_ref, w2_ref, b2_ref, out_ref):
    f = pl.program_id(1)
    x = buf_ref[0].astype(jnp.bfloat16)
    h = jnp.dot(x, w1_ref[0].astype(jnp.bfloat16),
                preferred_element_type=jnp.float32)
    h = h + b1_ref[0, 0]
    h = 0.5 * h * (1.0 + lax.erf(h * (1.0 / math.sqrt(2.0))))
    p = jnp.dot(h.astype(jnp.bfloat16), w2_ref[0].astype(jnp.bfloat16),
                preferred_element_type=jnp.float32)

    @pl.when(f == 0)
    def _():
        out_ref[0] = p + b2_ref[0, 0]

    @pl.when(f > 0)
    def _():
        out_ref[0] += p


def _expert_mlp(buf, W1, b1, W2, b2, C):
    nf = _F // _FT
    return pl.pallas_call(
        _mlp_body,
        grid=(_E, nf),
        in_specs=[
            pl.BlockSpec((1, C, _D), lambda e, f: (e, 0, 0)),
            pl.BlockSpec((1, _D, _FT), lambda e, f: (e, 0, f)),
            pl.BlockSpec((1, 1, _FT), lambda e, f: (e, 0, f)),
            pl.BlockSpec((1, _FT, _D), lambda e, f: (e, f, 0)),
            pl.BlockSpec((1, 1, _D), lambda e, f: (e, 0, 0)),
        ],
        out_specs=pl.BlockSpec((1, C, _D), lambda e, f: (e, 0, 0)),
        out_shape=jax.ShapeDtypeStruct((_E, C, _D), jnp.float32),
        compiler_params=pltpu.CompilerParams(
            dimension_semantics=("parallel", "arbitrary"),
        ),
    )(buf, W1, b1[:, None, :], W2, b2[:, None, :])


def kernel(input, Wg, W1, b1, W2, b2):
    B, S, Dm = input.shape
    T = B * S
    xf = input.reshape(T, Dm)
    C = int(_CAP_FACTOR * T * _TOP_K / _E)
    logits = xf @ Wg
    gates = jax.nn.softmax(logits, axis=-1)
    topv, topi = jax.lax.top_k(gates, _TOP_K)
    denom = jnp.sum(topv, axis=-1, keepdims=True) + 1e-9
    topw = topv / denom
    e_flat = topi.T.reshape(-1)
    w_flat = topw.T.reshape(-1)
    oh = jax.nn.one_hot(e_flat, _E, dtype=jnp.int32)
    pos_in_e = jnp.cumsum(oh, axis=0) - oh
    pos = jnp.sum(pos_in_e * oh, axis=1)
    keep = pos < C
    pos_c = jnp.where(keep, pos, 0)
    keep_f = keep.astype(xf.dtype)
    x_rep = jnp.tile(xf, (_TOP_K, 1))
    vals = x_rep * keep_f[:, None]
    buf = jnp.zeros((_E, C, Dm), dtype=xf.dtype).at[e_flat, pos_c].add(vals)
    eo = _expert_mlp(buf, W1, b1, W2, b2, C)
    gathered = eo[e_flat, pos_c]
    gathered = gathered * (keep_f * w_flat)[:, None]
    y = gathered.reshape(_TOP_K, T, Dm).sum(axis=0)
    out = y.reshape(B, S, Dm)
    aux = jnp.zeros((Dm,), dtype=input.dtype)
    return (out, aux)


# speedup vs baseline: 1.1571x; 1.0439x over previous
import functools
import math

import jax
import jax.numpy as jnp
from jax import lax
from jax.experimental import pallas as pl
from jax.experimental.pallas import tpu as pltpu

_E = 8
_TOP_K = 2
_D = 1024
_F = 4096
_CAP_FACTOR = 1.25

_FT = 2048


def _mlp_body(buf_ref, w1_ref, b1_ref, w2_ref, b2_ref, out_ref):
    f = pl.program_id(1)
    x = buf_ref[0].astype(jnp.bfloat16)
    h = jnp.dot(x, w1_ref[0].astype(jnp.bfloat16),
                preferred_element_type=jnp.float32)
    h = h + b1_ref[0, 0]
    h = 0.5 * h * (1.0 + lax.erf(h * (1.0 / math.sqrt(2.0))))
    p = jnp.dot(h.astype(jnp.bfloat16), w2_ref[0].astype(jnp.bfloat16),
                preferred_element_type=jnp.float32)

    @pl.when(f == 0)
    def _():
        out_ref[0] = p + b2_ref[0, 0]

    @pl.when(f > 0)
    def _():
        out_ref[0] += p


def _expert_mlp(buf, W1, b1, W2, b2, C):
    nf = _F // _FT
    return pl.pallas_call(
        _mlp_body,
        grid=(_E, nf),
        in_specs=[
            pl.BlockSpec((1, C, _D), lambda e, f: (e, 0, 0)),
            pl.BlockSpec((1, _D, _FT), lambda e, f: (e, 0, f)),
            pl.BlockSpec((1, 1, _FT), lambda e, f: (e, 0, f)),
            pl.BlockSpec((1, _FT, _D), lambda e, f: (e, f, 0)),
            pl.BlockSpec((1, 1, _D), lambda e, f: (e, 0, 0)),
        ],
        out_specs=pl.BlockSpec((1, C, _D), lambda e, f: (e, 0, 0)),
        out_shape=jax.ShapeDtypeStruct((_E, C, _D), jnp.float32),
        compiler_params=pltpu.CompilerParams(
            dimension_semantics=("parallel", "arbitrary"),
        ),
    )(buf, W1, b1[:, None, :], W2, b2[:, None, :])


def kernel(input, Wg, W1, b1, W2, b2):
    B, S, Dm = input.shape
    T = B * S
    xf = input.reshape(T, Dm)
    C = int(_CAP_FACTOR * T * _TOP_K / _E)
    logits = xf @ Wg
    gates = jax.nn.softmax(logits, axis=-1)
    topv, topi = jax.lax.top_k(gates, _TOP_K)
    denom = jnp.sum(topv, axis=-1, keepdims=True) + 1e-9
    topw = topv / denom
    e_flat = topi.T.reshape(-1)
    w_flat = topw.T.reshape(-1)
    oh = jax.nn.one_hot(e_flat, _E, dtype=jnp.int32)
    pos_in_e = jnp.cumsum(oh, axis=0) - oh
    pos = jnp.sum(pos_in_e * oh, axis=1)
    keep = pos < C
    pos_c = jnp.where(keep, pos, 0)
    keep_f = keep.astype(xf.dtype)
    x_rep = jnp.tile(xf, (_TOP_K, 1))
    vals = x_rep * keep_f[:, None]
    buf = jnp.zeros((_E, C, Dm), dtype=xf.dtype).at[e_flat, pos_c].add(vals)
    eo = _expert_mlp(buf, W1, b1, W2, b2, C)
    gathered = eo[e_flat, pos_c]
    gathered = gathered * (keep_f * w_flat)[:, None]
    y = gathered.reshape(_TOP_K, T, Dm).sum(axis=0)
    out = y.reshape(B, S, Dm)
    aux = jnp.zeros((Dm,), dtype=input.dtype)
    return (out, aux)


# speedup vs baseline: 1.3642x; 1.1790x over previous
import functools
import math

import jax
import jax.numpy as jnp
from jax import lax
from jax.experimental import pallas as pl
from jax.experimental.pallas import tpu as pltpu
from jax.experimental.pallas import tpu_sc as plsc

_E = 8
_TOP_K = 2
_D = 1024
_F = 4096
_T = 2048
_C = 640
_KT = _TOP_K * _T
_SB = 512
_FT = 2048

_NW = 32
_TRASH = _E * _C


def _route_body(x_ref, wg_ref, dst_ref, gidx_ref, wc_ref, base_ref):
    j = pl.program_id(0)
    k = j // (_T // _SB)

    @pl.when(j == 0)
    def _():
        base_ref[...] = jnp.zeros_like(base_ref)

    x = x_ref[...]
    logits = jnp.dot(x, wg_ref[...], preferred_element_type=jnp.float32)
    m = jnp.max(logits, axis=-1, keepdims=True)
    ex = jnp.exp(logits - m)
    gates = ex / jnp.sum(ex, axis=-1, keepdims=True)

    iota_e = lax.broadcasted_iota(jnp.int32, (_SB, _E), 1)
    m1 = jnp.max(gates, axis=-1, keepdims=True)
    i1 = jnp.min(jnp.where(gates == m1, iota_e, _E), axis=-1, keepdims=True)
    g2 = jnp.where(iota_e == i1, -jnp.inf, gates)
    m2 = jnp.max(g2, axis=-1, keepdims=True)
    i2 = jnp.min(jnp.where(g2 == m2, iota_e, _E), axis=-1, keepdims=True)
    denom = m1 + m2 + 1e-9
    w1 = m1 / denom
    w2 = m2 / denom

    first = (k == 0)
    e_sel = jnp.where(first, i1, i2)
    w_sel = jnp.where(first, w1, w2)

    oh = (iota_e == e_sel).astype(jnp.float32)
    ir = lax.broadcasted_iota(jnp.int32, (_SB, _SB), 0)
    ic = lax.broadcasted_iota(jnp.int32, (_SB, _SB), 1)
    stl = (ir > ic).astype(jnp.float32)
    prior = jnp.dot(stl, oh, preferred_element_type=jnp.float32)
    pos_all = base_ref[...] + prior
    pos = jnp.sum(pos_all * oh, axis=-1, keepdims=True).astype(jnp.int32)
    base_ref[...] += jnp.sum(oh, axis=0, keepdims=True)

    keep = pos < _C
    pos_c = jnp.where(keep, pos, 0)
    dst = jnp.where(keep, e_sel * _C + pos, _TRASH)
    gidx = e_sel * _C + pos_c
    wc = jnp.where(keep, w_sel, 0.0)

    dst_ref[0] = dst
    gidx_ref[0] = gidx
    wc_ref[0] = jnp.broadcast_to(wc, (_SB, 16))


def _route(xf, Wg):
    nblk = _KT // _SB
    return pl.pallas_call(
        _route_body,
        grid=(nblk,),
        in_specs=[
            pl.BlockSpec((_SB, _D), lambda j: (j % (_T // _SB), 0)),
            pl.BlockSpec((_D, _E), lambda j: (0, 0)),
        ],
        out_specs=[
            pl.BlockSpec((1, _SB, 1), lambda j: (j, 0, 0)),
            pl.BlockSpec((1, _SB, 1), lambda j: (j, 0, 0)),
            pl.BlockSpec((1, _SB, 16), lambda j: (j, 0, 0)),
        ],
        out_shape=[
            jax.ShapeDtypeStruct((nblk, _SB, 1), jnp.int32),
            jax.ShapeDtypeStruct((nblk, _SB, 1), jnp.int32),
            jax.ShapeDtypeStruct((nblk, _SB, 16), jnp.float32),
        ],
        scratch_shapes=[pltpu.VMEM((1, _E), jnp.float32)],
        compiler_params=pltpu.CompilerParams(
            dimension_semantics=("arbitrary",),
        ),
    )(xf, Wg)


@functools.lru_cache(maxsize=1)
def _sc_mesh():
    return plsc.VectorSubcoreMesh(core_axis_name="c", subcore_axis_name="s")


_CH = 32


def _dispatch_body(xf_hbm, dst_hbm, buf_hbm, idx0, idx1, rows0, rows1,
                   sem0, sem1):
    wid = lax.axis_index("s") * 2 + lax.axis_index("c")
    base = wid * (_KT // _NW)
    idxs = (idx0, idx1)
    rows = (rows0, rows1)
    sems = (sem0, sem1)
    handles = [None, None]
    for j in range(4):
        b = j % 2
        if handles[b] is not None:
            handles[b].wait()
        off = base + j * _CH
        src = lax.rem(off, _T)
        pltpu.sync_copy(dst_hbm.at[pl.ds(off, _CH)], idxs[b])
        pltpu.sync_copy(xf_hbm.at[pl.ds(src, _CH)], rows[b])
        handles[b] = pltpu.async_copy(rows[b], buf_hbm.at[idxs[b]], sems[b])
    handles[0].wait()
    handles[1].wait()


def _dispatch(xf, dst_flat):
    return pl.kernel(
        _dispatch_body,
        out_type=jax.ShapeDtypeStruct((_E * _C + 8, _D), jnp.float32),
        mesh=_sc_mesh(),
        scratch_types=[
            pltpu.VMEM((_CH,), jnp.int32),
            pltpu.VMEM((_CH,), jnp.int32),
            pltpu.VMEM((_CH, _D), jnp.float32),
            pltpu.VMEM((_CH, _D), jnp.float32),
            pltpu.SemaphoreType.DMA,
            pltpu.SemaphoreType.DMA,
        ],
    )(xf, dst_flat)


def _mlp_body(buf_ref, w1_ref, b1_ref, w2_ref, b2_ref, out_ref):
    f = pl.program_id(1)
    x = buf_ref[...].astype(jnp.bfloat16)
    h = jnp.dot(x, w1_ref[0].astype(jnp.bfloat16),
                preferred_element_type=jnp.float32)
    h = h + b1_ref[0, 0]
    h = 0.5 * h * (1.0 + lax.erf(h * (1.0 / math.sqrt(2.0))))
    p = jnp.dot(h.astype(jnp.bfloat16), w2_ref[0].astype(jnp.bfloat16),
                preferred_element_type=jnp.float32)

    @pl.when(f == 0)
    def _():
        out_ref[...] = p + b2_ref[0, 0]

    @pl.when(f > 0)
    def _():
        out_ref[...] += p


def _expert_mlp(buf, W1, b1, W2, b2):
    nf = _F // _FT
    return pl.pallas_call(
        _mlp_body,
        grid=(_E, nf),
        in_specs=[
            pl.BlockSpec((_C, _D), lambda e, f: (e, 0)),
            pl.BlockSpec((1, _D, _FT), lambda e, f: (e, 0, f)),
            pl.BlockSpec((1, 1, _FT), lambda e, f: (e, 0, f)),
            pl.BlockSpec((1, _FT, _D), lambda e, f: (e, f, 0)),
            pl.BlockSpec((1, 1, _D), lambda e, f: (e, 0, 0)),
        ],
        out_specs=pl.BlockSpec((_C, _D), lambda e, f: (e, 0)),
        out_shape=jax.ShapeDtypeStruct((_E * _C, _D), jnp.float32),
        compiler_params=pltpu.CompilerParams(
            dimension_semantics=("parallel", "arbitrary"),
        ),
    )(buf, W1, b1[:, None, :], W2, b2[:, None, :])


_CT = 32


def _combine_body(eo_hbm, gidx_hbm, wc_hbm, y_hbm, idxa, idxb, wa, wb,
                  rowsa, rowsb, yv, sema, semb):
    wid = lax.axis_index("s") * 2 + lax.axis_index("c")
    base = wid * (_T // _NW)
    for ch in range(2):
        t0 = base + ch * _CT
        pltpu.sync_copy(gidx_hbm.at[pl.ds(t0, _CT)], idxa)
        pltpu.sync_copy(gidx_hbm.at[pl.ds(_T + t0, _CT)], idxb)
        pltpu.sync_copy(wc_hbm.at[pl.ds(t0, _CT)], wa)
        pltpu.sync_copy(wc_hbm.at[pl.ds(_T + t0, _CT)], wb)
        ha = pltpu.async_copy(eo_hbm.at[idxa], rowsa, sema)
        hb = pltpu.async_copy(eo_hbm.at[idxb], rowsb, semb)
        ha.wait()
        hb.wait()

        def _row(i, carry):
            wai = wa[i, pl.ds(0, 16)]
            wbi = wb[i, pl.ds(0, 16)]
            for c in range(_D // 16):
                sl = pl.ds(c * 16, 16)
                yv[i, sl] = rowsa[i, sl] * wai + rowsb[i, sl] * wbi
            return carry

        lax.fori_loop(0, _CT, _row, 0)
        pltpu.sync_copy(yv, y_hbm.at[pl.ds(t0, _CT)])


def _combine(eo, gidx_flat, wc_flat):
    return pl.kernel(
        _combine_body,
        out_type=jax.ShapeDtypeStruct((_T, _D), jnp.float32),
        mesh=_sc_mesh(),
        scratch_types=[
            pltpu.VMEM((_CT,), jnp.int32),
            pltpu.VMEM((_CT,), jnp.int32),
            pltpu.VMEM((_CT, 16), jnp.float32),
            pltpu.VMEM((_CT, 16), jnp.float32),
            pltpu.VMEM((_CT, _D), jnp.float32),
            pltpu.VMEM((_CT, _D), jnp.float32),
            pltpu.VMEM((_CT, _D), jnp.float32),
            pltpu.SemaphoreType.DMA,
            pltpu.SemaphoreType.DMA,
        ],
    )(eo, gidx_flat, wc_flat)


def kernel(input, Wg, W1, b1, W2, b2):
    B, S, Dm = input.shape
    xf = input.reshape(B * S, Dm)
    dst, gidx, wc = _route(xf, Wg)
    dst_flat = dst.reshape(_KT)
    gidx_flat = gidx.reshape(_KT)
    wc_flat = wc.reshape(_KT, 16)
    buf = _dispatch(xf, dst_flat)
    eo = _expert_mlp(buf, W1, b1, W2, b2)
    y = _combine(eo, gidx_flat, wc_flat)
    out = y.reshape(B, S, Dm)
    aux = jnp.zeros((Dm,), dtype=input.dtype)
    return (out, aux)


# speedup vs baseline: 1.3986x; 1.0252x over previous
import functools
import math

import jax
import jax.numpy as jnp
from jax import lax
from jax.experimental import pallas as pl
from jax.experimental.pallas import tpu as pltpu
from jax.experimental.pallas import tpu_sc as plsc

_E = 8
_TOP_K = 2
_D = 1024
_F = 4096
_T = 2048
_C = 640
_KT = _TOP_K * _T
_SB = 512
_FT = 2048

_NW = 32
_TRASH = _E * _C


def _route_body(x_ref, wg_ref, dst_ref, gidx_ref, wc_ref, base_ref):
    j = pl.program_id(0)
    k = j // (_T // _SB)

    @pl.when(j == 0)
    def _():
        base_ref[...] = jnp.zeros_like(base_ref)

    x = x_ref[...]
    logits = jnp.dot(x, wg_ref[...], preferred_element_type=jnp.float32)
    m = jnp.max(logits, axis=-1, keepdims=True)
    ex = jnp.exp(logits - m)
    gates = ex / jnp.sum(ex, axis=-1, keepdims=True)

    iota_e = lax.broadcasted_iota(jnp.int32, (_SB, _E), 1)
    m1 = jnp.max(gates, axis=-1, keepdims=True)
    i1 = jnp.min(jnp.where(gates == m1, iota_e, _E), axis=-1, keepdims=True)
    g2 = jnp.where(iota_e == i1, -jnp.inf, gates)
    m2 = jnp.max(g2, axis=-1, keepdims=True)
    i2 = jnp.min(jnp.where(g2 == m2, iota_e, _E), axis=-1, keepdims=True)
    denom = m1 + m2 + 1e-9
    w1 = m1 / denom
    w2 = m2 / denom

    first = (k == 0)
    e_sel = jnp.where(first, i1, i2)
    w_sel = jnp.where(first, w1, w2)

    oh = (iota_e == e_sel).astype(jnp.float32)
    ir = lax.broadcasted_iota(jnp.int32, (_SB, _SB), 0)
    ic = lax.broadcasted_iota(jnp.int32, (_SB, _SB), 1)
    stl = (ir > ic).astype(jnp.float32)
    prior = jnp.dot(stl, oh, preferred_element_type=jnp.float32)
    pos_all = base_ref[...] + prior
    pos = jnp.sum(pos_all * oh, axis=-1, keepdims=True).astype(jnp.int32)
    base_ref[...] += jnp.sum(oh, axis=0, keepdims=True)

    keep = pos < _C
    pos_c = jnp.where(keep, pos, 0)
    dst = jnp.where(keep, e_sel * _C + pos, _TRASH)
    gidx = e_sel * _C + pos_c
    wc = jnp.where(keep, w_sel, 0.0)

    dst_ref[0] = dst
    gidx_ref[0] = gidx
    wc_ref[0] = jnp.broadcast_to(wc, (_SB, 16))


def _route(xf, Wg):
    nblk = _KT // _SB
    return pl.pallas_call(
        _route_body,
        grid=(nblk,),
        in_specs=[
            pl.BlockSpec((_SB, _D), lambda j: (j % (_T // _SB), 0)),
            pl.BlockSpec((_D, _E), lambda j: (0, 0)),
        ],
        out_specs=[
            pl.BlockSpec((1, _SB, 1), lambda j: (j, 0, 0)),
            pl.BlockSpec((1, _SB, 1), lambda j: (j, 0, 0)),
            pl.BlockSpec((1, _SB, 16), lambda j: (j, 0, 0)),
        ],
        out_shape=[
            jax.ShapeDtypeStruct((nblk, _SB, 1), jnp.int32),
            jax.ShapeDtypeStruct((nblk, _SB, 1), jnp.int32),
            jax.ShapeDtypeStruct((nblk, _SB, 16), jnp.float32),
        ],
        scratch_shapes=[pltpu.VMEM((1, _E), jnp.float32)],
        compiler_params=pltpu.CompilerParams(
            dimension_semantics=("arbitrary",),
        ),
    )(xf, Wg)


@functools.lru_cache(maxsize=1)
def _sc_mesh():
    return plsc.VectorSubcoreMesh(core_axis_name="c", subcore_axis_name="s")


_CH = 32


def _dispatch_body(xf_hbm, dst_hbm, buf_hbm, idx0, idx1, idx2, idx3,
                   rows0, rows1, semi0, semi1, semo0, semo1):
    wid = lax.axis_index("s") * 2 + lax.axis_index("c")
    base = wid * (_KT // _NW)
    idxs = (idx0, idx1, idx2, idx3)
    rows = (rows0, rows1)
    semi = (semi0, semi1)
    semo = (semo0, semo1)
    for j in range(4):
        pltpu.sync_copy(dst_hbm.at[pl.ds(base + j * _CH, _CH)], idxs[j])

    def _start_in(j, b):
        src = lax.rem(base + j * _CH, _T)
        return pltpu.async_copy(xf_hbm.at[pl.ds(src, _CH)], rows[b], semi[b])

    h_in = [_start_in(0, 0), _start_in(1, 1)]
    h_out = [None, None]
    for j in range(4):
        b = j % 2
        h_in[b].wait()
        h_out[b] = pltpu.async_copy(rows[b], buf_hbm.at[idxs[j]], semo[b])
        if j + 2 < 4:
            h_out[b].wait()
            h_in[b] = _start_in(j + 2, b)
    h_out[0].wait()
    h_out[1].wait()


def _dispatch(xf, dst_flat):
    return pl.kernel(
        _dispatch_body,
        out_type=jax.ShapeDtypeStruct((_E * _C + 8, _D), jnp.float32),
        mesh=_sc_mesh(),
        scratch_types=[
            pltpu.VMEM((_CH,), jnp.int32),
            pltpu.VMEM((_CH,), jnp.int32),
            pltpu.VMEM((_CH,), jnp.int32),
            pltpu.VMEM((_CH,), jnp.int32),
            pltpu.VMEM((_CH, _D), jnp.float32),
            pltpu.VMEM((_CH, _D), jnp.float32),
            pltpu.SemaphoreType.DMA,
            pltpu.SemaphoreType.DMA,
            pltpu.SemaphoreType.DMA,
            pltpu.SemaphoreType.DMA,
        ],
    )(xf, dst_flat)


def _mlp_body(buf_ref, w1_ref, b1_ref, w2_ref, b2_ref, out_ref):
    f = pl.program_id(1)
    x = buf_ref[...].astype(jnp.bfloat16)
    h = jnp.dot(x, w1_ref[0].astype(jnp.bfloat16),
                preferred_element_type=jnp.float32)
    h = h + b1_ref[0, 0]
    h = 0.5 * h * (1.0 + lax.erf(h * (1.0 / math.sqrt(2.0))))
    p = jnp.dot(h.astype(jnp.bfloat16), w2_ref[0].astype(jnp.bfloat16),
                preferred_element_type=jnp.float32)

    @pl.when(f == 0)
    def _():
        out_ref[...] = p + b2_ref[0, 0]

    @pl.when(f > 0)
    def _():
        out_ref[...] += p


def _expert_mlp(buf, W1, b1, W2, b2):
    nf = _F // _FT
    return pl.pallas_call(
        _mlp_body,
        grid=(_E, nf),
        in_specs=[
            pl.BlockSpec((_C, _D), lambda e, f: (e, 0)),
            pl.BlockSpec((1, _D, _FT), lambda e, f: (e, 0, f)),
            pl.BlockSpec((1, 1, _FT), lambda e, f: (e, 0, f)),
            pl.BlockSpec((1, _FT, _D), lambda e, f: (e, f, 0)),
            pl.BlockSpec((1, 1, _D), lambda e, f: (e, 0, 0)),
        ],
        out_specs=pl.BlockSpec((_C, _D), lambda e, f: (e, 0)),
        out_shape=jax.ShapeDtypeStruct((_E * _C, _D), jnp.float32),
        compiler_params=pltpu.CompilerParams(
            dimension_semantics=("parallel", "arbitrary"),
        ),
    )(buf, W1, b1[:, None, :], W2, b2[:, None, :])


_CT = 16


def _combine_body(eo_hbm, gidx_hbm, wc_hbm, y_hbm,
                  idxa0, idxb0, idxa1, idxb1, idxa2, idxb2, idxa3, idxb3,
                  wa, wb, rowsa0, rowsb0, rowsa1, rowsb1, yv0, yv1,
                  sema0, semb0, sema1, semb1, semy0, semy1):
    wid = lax.axis_index("s") * 2 + lax.axis_index("c")
    base = wid * (_T // _NW)
    nch = (_T // _NW) // _CT
    idxa = (idxa0, idxa1, idxa2, idxa3)
    idxb = (idxb0, idxb1, idxb2, idxb3)
    rowsa = (rowsa0, rowsa1)
    rowsb = (rowsb0, rowsb1)
    yv = (yv0, yv1)
    sema = (sema0, sema1)
    semb = (semb0, semb1)
    semy = (semy0, semy1)

    pltpu.sync_copy(wc_hbm.at[pl.ds(base, _T // _NW)], wa)
    pltpu.sync_copy(wc_hbm.at[pl.ds(_T + base, _T // _NW)], wb)
    for ch in range(nch):
        pltpu.sync_copy(gidx_hbm.at[pl.ds(base + ch * _CT, _CT)], idxa[ch])
        pltpu.sync_copy(gidx_hbm.at[pl.ds(_T + base + ch * _CT, _CT)],
                        idxb[ch])

    def _start_gather(ch, b):
        ha = pltpu.async_copy(eo_hbm.at[idxa[ch]], rowsa[b], sema[b])
        hb = pltpu.async_copy(eo_hbm.at[idxb[ch]], rowsb[b], semb[b])
        return (ha, hb)

    h_g = [_start_gather(0, 0), _start_gather(1, 1)]
    h_y = [None, None]
    for ch in range(nch):
        b = ch % 2
        h_g[b][0].wait()
        h_g[b][1].wait()
        if h_y[b] is not None:
            h_y[b].wait()

        def _row(i, carry):
            wai = wa[ch * _CT + i, pl.ds(0, 16)]
            wbi = wb[ch * _CT + i, pl.ds(0, 16)]
            for c in range(_D // 16):
                sl = pl.ds(c * 16, 16)
                yv[b][i, sl] = rowsa[b][i, sl] * wai + rowsb[b][i, sl] * wbi
            return carry

        lax.fori_loop(0, _CT, _row, 0)
        h_y[b] = pltpu.async_copy(yv[b], y_hbm.at[pl.ds(base + ch * _CT,
                                                        _CT)], semy[b])
        if ch + 2 < nch:
            h_g[b] = _start_gather(ch + 2, b)
    h_y[0].wait()
    h_y[1].wait()


def _combine(eo, gidx_flat, wc_flat):
    return pl.kernel(
        _combine_body,
        out_type=jax.ShapeDtypeStruct((_T, _D), jnp.float32),
        mesh=_sc_mesh(),
        scratch_types=(
            [pltpu.VMEM((_CT,), jnp.int32)] * 8
            + [pltpu.VMEM((_T // _NW, 16), jnp.float32)] * 2
            + [pltpu.VMEM((_CT, _D), jnp.float32)] * 6
            + [pltpu.SemaphoreType.DMA] * 6
        ),
    )(eo, gidx_flat, wc_flat)


def kernel(input, Wg, W1, b1, W2, b2):
    B, S, Dm = input.shape
    xf = input.reshape(B * S, Dm)
    dst, gidx, wc = _route(xf, Wg)
    dst_flat = dst.reshape(_KT)
    gidx_flat = gidx.reshape(_KT)
    wc_flat = wc.reshape(_KT, 16)
    buf = _dispatch(xf, dst_flat)
    eo = _expert_mlp(buf, W1, b1, W2, b2)
    y = _combine(eo, gidx_flat, wc_flat)
    out = y.reshape(B, S, Dm)
    aux = jnp.zeros((Dm,), dtype=input.dtype)
    return (out, aux)


# speedup vs baseline: 1.4103x; 1.0083x over previous
import functools
import math

import jax
import jax.numpy as jnp
from jax import lax
from jax.experimental import pallas as pl
from jax.experimental.pallas import tpu as pltpu
from jax.experimental.pallas import tpu_sc as plsc

_E = 8
_TOP_K = 2
_D = 1024
_F = 4096
_T = 2048
_C = 640
_KT = _TOP_K * _T
_SB = 512
_FT = 2048

_NW = 32
_TRASH = _E * _C


def _route_body(x_ref, wg_ref, dst_ref, gidx_ref, wc_ref, base_ref):
    j = pl.program_id(0)
    k = j // (_T // _SB)

    @pl.when(j == 0)
    def _():
        base_ref[...] = jnp.zeros_like(base_ref)

    x = x_ref[...]
    logits = jnp.dot(x, wg_ref[...], preferred_element_type=jnp.float32)
    m = jnp.max(logits, axis=-1, keepdims=True)
    ex = jnp.exp(logits - m)
    gates = ex / jnp.sum(ex, axis=-1, keepdims=True)

    iota_e = lax.broadcasted_iota(jnp.int32, (_SB, _E), 1)
    m1 = jnp.max(gates, axis=-1, keepdims=True)
    i1 = jnp.min(jnp.where(gates == m1, iota_e, _E), axis=-1, keepdims=True)
    g2 = jnp.where(iota_e == i1, -jnp.inf, gates)
    m2 = jnp.max(g2, axis=-1, keepdims=True)
    i2 = jnp.min(jnp.where(g2 == m2, iota_e, _E), axis=-1, keepdims=True)
    denom = m1 + m2 + 1e-9
    w1 = m1 / denom
    w2 = m2 / denom

    first = (k == 0)
    e_sel = jnp.where(first, i1, i2)
    w_sel = jnp.where(first, w1, w2)

    oh = (iota_e == e_sel).astype(jnp.float32)
    ir = lax.broadcasted_iota(jnp.int32, (_SB, _SB), 0)
    ic = lax.broadcasted_iota(jnp.int32, (_SB, _SB), 1)
    stl = (ir > ic).astype(jnp.float32)
    prior = jnp.dot(stl, oh, preferred_element_type=jnp.float32)
    pos_all = base_ref[...] + prior
    pos = jnp.sum(pos_all * oh, axis=-1, keepdims=True).astype(jnp.int32)
    base_ref[...] += jnp.sum(oh, axis=0, keepdims=True)

    keep = pos < _C
    pos_c = jnp.where(keep, pos, 0)
    dst = jnp.where(keep, e_sel * _C + pos, _TRASH)
    gidx = e_sel * _C + pos_c
    wc = jnp.where(keep, w_sel, 0.0)

    dst_ref[0] = dst
    gidx_ref[0] = gidx
    wc_ref[0] = jnp.broadcast_to(wc, (_SB, 16))


def _route(xf, Wg):
    nblk = _KT // _SB
    return pl.pallas_call(
        _route_body,
        grid=(nblk,),
        in_specs=[
            pl.BlockSpec((_SB, _D), lambda j: (j % (_T // _SB), 0)),
            pl.BlockSpec((_D, _E), lambda j: (0, 0)),
        ],
        out_specs=[
            pl.BlockSpec((1, _SB, 1), lambda j: (j, 0, 0)),
            pl.BlockSpec((1, _SB, 1), lambda j: (j, 0, 0)),
            pl.BlockSpec((1, _SB, 16), lambda j: (j, 0, 0)),
        ],
        out_shape=[
            jax.ShapeDtypeStruct((nblk, _SB, 1), jnp.int32),
            jax.ShapeDtypeStruct((nblk, _SB, 1), jnp.int32),
            jax.ShapeDtypeStruct((nblk, _SB, 16), jnp.float32),
        ],
        scratch_shapes=[pltpu.VMEM((1, _E), jnp.float32)],
        compiler_params=pltpu.CompilerParams(
            dimension_semantics=("arbitrary",),
        ),
    )(xf, Wg)


@functools.lru_cache(maxsize=1)
def _sc_mesh():
    return plsc.VectorSubcoreMesh(core_axis_name="c", subcore_axis_name="s")


_CH = 32


def _dispatch_body(xf_hbm, dst2_hbm, buf_hbm, idx4, rows0, rows1, rows2,
                   semi0, semi1, semi2, semo0, semo1, semo2):
    wid = lax.axis_index("s") * 2 + lax.axis_index("c")
    base = wid * (_KT // _NW)
    rows = (rows0, rows1, rows2)
    semi = (semi0, semi1, semi2)
    semo = (semo0, semo1, semo2)

    def _start_in(j, b):
        src = lax.rem(base + j * _CH, _T)
        return pltpu.async_copy(xf_hbm.at[pl.ds(src, _CH)], rows[b], semi[b])

    h_in = [_start_in(0, 0), _start_in(1, 1), _start_in(2, 2)]
    pltpu.sync_copy(dst2_hbm.at[pl.ds(wid * 4, 4)], idx4)
    h_out = [None, None, None]
    for j in range(4):
        b = j % 3
        h_in[b].wait()
        h_out[b] = pltpu.async_copy(rows[b], buf_hbm.at[idx4.at[j]], semo[b])
        if j == 1:
            h_out[0].wait()
            h_in[0] = _start_in(3, 0)
    h_out[0].wait()
    h_out[1].wait()
    h_out[2].wait()


def _dispatch(xf, dst2):
    return pl.kernel(
        _dispatch_body,
        out_type=jax.ShapeDtypeStruct((_E * _C + 8, _D), jnp.float32),
        mesh=_sc_mesh(),
        scratch_types=[
            pltpu.VMEM((4, _CH), jnp.int32),
            pltpu.VMEM((_CH, _D), jnp.float32),
            pltpu.VMEM((_CH, _D), jnp.float32),
            pltpu.VMEM((_CH, _D), jnp.float32),
            pltpu.SemaphoreType.DMA,
            pltpu.SemaphoreType.DMA,
            pltpu.SemaphoreType.DMA,
            pltpu.SemaphoreType.DMA,
            pltpu.SemaphoreType.DMA,
            pltpu.SemaphoreType.DMA,
        ],
    )(xf, dst2)


def _mlp_body(buf_ref, w1_ref, b1_ref, w2_ref, b2_ref, out_ref):
    f = pl.program_id(1)
    x = buf_ref[...].astype(jnp.bfloat16)
    h = jnp.dot(x, w1_ref[0].astype(jnp.bfloat16),
                preferred_element_type=jnp.float32)
    h = h + b1_ref[0, 0]
    h = 0.5 * h * (1.0 + lax.erf(h * (1.0 / math.sqrt(2.0))))
    p = jnp.dot(h.astype(jnp.bfloat16), w2_ref[0].astype(jnp.bfloat16),
                preferred_element_type=jnp.float32)

    @pl.when(f == 0)
    def _():
        out_ref[...] = p + b2_ref[0, 0]

    @pl.when(f > 0)
    def _():
        out_ref[...] += p


def _expert_mlp(buf, W1, b1, W2, b2):
    nf = _F // _FT
    return pl.pallas_call(
        _mlp_body,
        grid=(_E, nf),
        in_specs=[
            pl.BlockSpec((_C, _D), lambda e, f: (e, 0)),
            pl.BlockSpec((1, _D, _FT), lambda e, f: (e, 0, f)),
            pl.BlockSpec((1, 1, _FT), lambda e, f: (e, 0, f)),
            pl.BlockSpec((1, _FT, _D), lambda e, f: (e, f, 0)),
            pl.BlockSpec((1, 1, _D), lambda e, f: (e, 0, 0)),
        ],
        out_specs=pl.BlockSpec((_C, _D), lambda e, f: (e, 0)),
        out_shape=jax.ShapeDtypeStruct((_E * _C, _D), jnp.float32),
        compiler_params=pltpu.CompilerParams(
            dimension_semantics=("parallel", "arbitrary"),
        ),
    )(buf, W1, b1[:, None, :], W2, b2[:, None, :])


_CT = 16


def _combine_body(eo_hbm, gidx_hbm, wc_hbm, y_hbm,
                  idxa0, idxb0, idxa1, idxb1, idxa2, idxb2, idxa3, idxb3,
                  wa, wb, rowsa0, rowsb0, rowsa1, rowsb1, yv0, yv1,
                  sema0, semb0, sema1, semb1, semy0, semy1):
    wid = lax.axis_index("s") * 2 + lax.axis_index("c")
    base = wid * (_T // _NW)
    nch = (_T // _NW) // _CT
    idxa = (idxa0, idxa1, idxa2, idxa3)
    idxb = (idxb0, idxb1, idxb2, idxb3)
    rowsa = (rowsa0, rowsa1)
    rowsb = (rowsb0, rowsb1)
    yv = (yv0, yv1)
    sema = (sema0, sema1)
    semb = (semb0, semb1)
    semy = (semy0, semy1)

    pltpu.sync_copy(wc_hbm.at[pl.ds(base, _T // _NW)], wa)
    pltpu.sync_copy(wc_hbm.at[pl.ds(_T + base, _T // _NW)], wb)
    for ch in range(nch):
        pltpu.sync_copy(gidx_hbm.at[pl.ds(base + ch * _CT, _CT)], idxa[ch])
        pltpu.sync_copy(gidx_hbm.at[pl.ds(_T + base + ch * _CT, _CT)],
                        idxb[ch])

    def _start_gather(ch, b):
        ha = pltpu.async_copy(eo_hbm.at[idxa[ch]], rowsa[b], sema[b])
        hb = pltpu.async_copy(eo_hbm.at[idxb[ch]], rowsb[b], semb[b])
        return (ha, hb)

    h_g = [_start_gather(0, 0), _start_gather(1, 1)]
    h_y = [None, None]
    for ch in range(nch):
        b = ch % 2
        h_g[b][0].wait()
        h_g[b][1].wait()
        if h_y[b] is not None:
            h_y[b].wait()

        def _row(i, carry):
            wai = wa[ch * _CT + i, pl.ds(0, 16)]
            wbi = wb[ch * _CT + i, pl.ds(0, 16)]
            for c in range(_D // 16):
                sl = pl.ds(c * 16, 16)
                yv[b][i, sl] = rowsa[b][i, sl] * wai + rowsb[b][i, sl] * wbi
            return carry

        lax.fori_loop(0, _CT, _row, 0)
        h_y[b] = pltpu.async_copy(yv[b], y_hbm.at[pl.ds(base + ch * _CT,
                                                        _CT)], semy[b])
        if ch + 2 < nch:
            h_g[b] = _start_gather(ch + 2, b)
    h_y[0].wait()
    h_y[1].wait()


def _combine(eo, gidx_flat, wc_flat):
    return pl.kernel(
        _combine_body,
        out_type=jax.ShapeDtypeStruct((_T, _D), jnp.float32),
        mesh=_sc_mesh(),
        scratch_types=(
            [pltpu.VMEM((_CT,), jnp.int32)] * 8
            + [pltpu.VMEM((_T // _NW, 16), jnp.float32)] * 2
            + [pltpu.VMEM((_CT, _D), jnp.float32)] * 6
            + [pltpu.SemaphoreType.DMA] * 6
        ),
    )(eo, gidx_flat, wc_flat)


def kernel(input, Wg, W1, b1, W2, b2):
    B, S, Dm = input.shape
    xf = input.reshape(B * S, Dm)
    dst, gidx, wc = _route(xf, Wg)
    dst2 = dst.reshape(_KT // _CH, _CH)
    gidx_flat = gidx.reshape(_KT)
    wc_flat = wc.reshape(_KT, 16)
    buf = _dispatch(xf, dst2)
    eo = _expert_mlp(buf, W1, b1, W2, b2)
    y = _combine(eo, gidx_flat, wc_flat)
    out = y.reshape(B, S, Dm)
    aux = jnp.zeros((Dm,), dtype=input.dtype)
    return (out, aux)
